# Initial kernel scaffold; baseline (speedup 1.0000x reference)
import jax
import jax.numpy as jnp
from jax.experimental import pallas as pl


def kernel(x, gate_W, gate_b, expert_W, expert_b):
    raise NotImplementedError("write your pallas kernel here")


# speedup vs baseline: 2.1664x; 2.1664x over previous
import jax
import jax.numpy as jnp
from jax.experimental import pallas as pl
from jax.experimental.pallas import tpu as pltpu


def _moe_block_kernel(x_ref, gw_ref, gb_ref, w_ref, b_ref, o_ref, *, n_exp):
    j = pl.program_id(1)
    xb = x_ref[...].astype(jnp.bfloat16)
    w = w_ref[0].astype(jnp.bfloat16)
    y = jax.lax.dot_general(
        xb, w, (((1,), (1,)), ((), ())), preferred_element_type=jnp.float32
    )
    y = jnp.maximum(y + b_ref[0], 0.0)

    @pl.when(j == 0)
    def _():
        o_ref[...] = y

    @pl.when(j > 0)
    def _():
        o_ref[...] += y

    @pl.when(j == n_exp - 1)
    def _():
        gw = gw_ref[...].astype(jnp.bfloat16)
        logits = jax.lax.dot_general(
            xb, gw, (((1,), (1,)), ((), ())), preferred_element_type=jnp.float32
        ) + gb_ref[...]
        p = jax.nn.softmax(logits, axis=-1)
        m1 = jnp.max(p, axis=-1, keepdims=True)
        lane = jax.lax.broadcasted_iota(jnp.int32, p.shape, 1)
        first = jnp.min(
            jnp.where(p == m1, lane, p.shape[-1]), axis=-1, keepdims=True
        )
        m2 = jnp.max(jnp.where(lane == first, -1.0, p), axis=-1, keepdims=True)
        o_ref[...] *= m1 + m2


def kernel(x, gate_W, gate_b, expert_W, expert_b):
    B, S, D = x.shape
    E = gate_W.shape[0]
    T = B * S
    blk = 2048
    n_tblk = T // blk

    xf = x.reshape(T, D)
    gb2 = gate_b.reshape(1, E)
    eb3 = expert_b.reshape(E, 1, D)

    import functools

    out = pl.pallas_call(
        functools.partial(_moe_block_kernel, n_exp=E),
        grid=(n_tblk, E),
        in_specs=[
            pl.BlockSpec((blk, D), lambda t, j: (t, 0)),
            pl.BlockSpec((E, D), lambda t, j: (0, 0)),
            pl.BlockSpec((1, E), lambda t, j: (0, 0)),
            pl.BlockSpec((1, D, D), lambda t, j: (j, 0, 0)),
            pl.BlockSpec((1, 1, D), lambda t, j: (j, 0, 0)),
        ],
        out_specs=pl.BlockSpec((blk, D), lambda t, j: (t, 0)),
        out_shape=jax.ShapeDtypeStruct((T, D), jnp.float32),
        compiler_params=pltpu.CompilerParams(
            dimension_semantics=("parallel", "arbitrary")
        ),
    )(xf, gate_W, gb2, expert_W, eb3)
    return out.reshape(B, S, D)
